# baseline (speedup 1.0000x reference)
import jax
import jax.numpy as jnp
import numpy as np
from jax.experimental import pallas as pl
from jax.experimental.pallas import tpu as pltpu

BN_EPS = 1e-5


def _conv_stats_kernel(x_ref, w4_ref, sel_ref, y_ref, psum_ref, psq_ref, *,
                       n_pairs, w, c, cout):
    w4 = w4_ref[...]
    sel = sel_ref[...]
    w2h = w // 2
    acc_s = jnp.zeros((w, cout), jnp.float32)
    acc_q = jnp.zeros((w, cout), jnp.float32)
    for u in range(n_pairs // 2):
        rows = [x_ref[0, :, (u * 4 + k) * w:(u * 4 + k + 1) * w]
                for k in range(4)]
        xq4 = jnp.concatenate(rows, axis=0)
        d = jnp.dot(xq4, sel, preferred_element_type=jnp.float32)
        a0, b0 = d[0:c, :w2h], d[0:c, w2h:]
        c0, d0 = d[c:2 * c, :w2h], d[c:2 * c, w2h:]
        a1, b1 = d[2 * c:3 * c, :w2h], d[2 * c:3 * c, w2h:]
        c1, d1 = d[3 * c:, :w2h], d[3 * c:, w2h:]
        cat = jnp.concatenate([
            jnp.concatenate([a0, a1], axis=1),
            jnp.concatenate([b0, b1], axis=1),
            jnp.concatenate([c0, c1], axis=1),
            jnp.concatenate([d0, d1], axis=1),
        ], axis=0)
        y2 = jax.lax.dot_general(
            cat, w4, (((0,), (1,)), ((), ())),
            preferred_element_type=jnp.float32)
        y_ref[0, u * w:(u + 1) * w, :] = y2.astype(y_ref.dtype)
        acc_s = acc_s + y2
        acc_q = acc_q + y2 * y2
    psum_ref[0] = jnp.sum(acc_s, axis=0, keepdims=True)
    psq_ref[0] = jnp.sum(acc_q, axis=0, keepdims=True)


def _bn_relu_kernel(y_ref, scale_ref, shift_ref, o_ref):
    y = y_ref[0].astype(jnp.float32)
    o_ref[0] = jnp.maximum(y * scale_ref[...] + shift_ref[...], 0.0)


def kernel(x, conv_w, conv_b, bn_gamma, bn_beta):
    N, C, H, W = x.shape
    H2, W2 = H // 2, W // 2
    HW = H * W
    P = N * H2 * W2
    Cout = conv_w.shape[0]

    wf = conv_w.reshape(Cout, 4 * C).astype(jnp.float32)
    w0, w1, w2_, w3 = (wf[:, 0:C], wf[:, C:2 * C],
                       wf[:, 2 * C:3 * C], wf[:, 3 * C:])
    wa = 0.5 * (w0 + w1 + w2_ + w3)
    wb = 0.5 * (w0 + w1 - w2_ - w3)
    wc = 0.5 * (w0 - w1 + w2_ - w3)
    wd = 0.5 * (w0 - w1 - w2_ + w3)
    w4 = jnp.concatenate([wa, wb, wc, wd], axis=1)

    W2h = W // 2
    sel_np = np.zeros((W, W), np.float32)
    sel_np[2 * np.arange(W2h), np.arange(W2h)] = 1.0
    sel_np[2 * np.arange(W2h) + 1, W2h + np.arange(W2h)] = 1.0
    sel = jnp.asarray(sel_np)

    npb = H2 // 2 if H2 % 2 == 0 else H2
    LB = npb * 2 * W
    J = HW // LB

    xr = x.reshape(N, C, HW)

    cparams = pltpu.CompilerParams(
        dimension_semantics=("parallel", "parallel"),
        vmem_limit_bytes=100 * 1024 * 1024,
    )

    y, psum, psq = pl.pallas_call(
        lambda *refs: _conv_stats_kernel(*refs, n_pairs=npb, w=W, c=C,
                                         cout=Cout),
        grid=(N, J),
        in_specs=[
            pl.BlockSpec((1, C, LB), lambda n, j: (n, 0, j)),
            pl.BlockSpec((Cout, 4 * C), lambda n, j: (0, 0)),
            pl.BlockSpec((W, W), lambda n, j: (0, 0)),
        ],
        out_specs=[
            pl.BlockSpec((1, LB // 4, Cout), lambda n, j: (n, j, 0)),
            pl.BlockSpec((1, 1, Cout), lambda n, j: (n * J + j, 0, 0)),
            pl.BlockSpec((1, 1, Cout), lambda n, j: (n * J + j, 0, 0)),
        ],
        out_shape=[
            jax.ShapeDtypeStruct((N, HW // 4, Cout), jnp.bfloat16),
            jax.ShapeDtypeStruct((N * J, 1, Cout), jnp.float32),
            jax.ShapeDtypeStruct((N * J, 1, Cout), jnp.float32),
        ],
        compiler_params=cparams,
    )(xr, w4, sel)

    conv_b = conv_b.astype(jnp.float32)
    sum_y = jnp.sum(psum[:, 0, :], axis=0)
    sum_y2 = jnp.sum(psq[:, 0, :], axis=0)
    mean_nb = sum_y / P
    var = jnp.maximum(sum_y2 / P - mean_nb * mean_nb, 0.0)
    inv_std = jax.lax.rsqrt(var + BN_EPS)
    scale = bn_gamma.astype(jnp.float32) * inv_std
    mean_wb = mean_nb + conv_b
    shift = bn_beta.astype(jnp.float32) + scale * (conv_b - mean_wb)
    scale2 = scale.reshape(1, Cout)
    shift2 = shift.reshape(1, Cout)

    P2 = HW // 4
    out = pl.pallas_call(
        _bn_relu_kernel,
        grid=(N,),
        in_specs=[
            pl.BlockSpec((1, P2, Cout), lambda n: (n, 0, 0)),
            pl.BlockSpec((1, Cout), lambda n: (0, 0)),
            pl.BlockSpec((1, Cout), lambda n: (0, 0)),
        ],
        out_specs=pl.BlockSpec((1, P2, Cout), lambda n: (n, 0, 0)),
        out_shape=jax.ShapeDtypeStruct((N, P2, Cout), jnp.float32),
        compiler_params=pltpu.CompilerParams(
            dimension_semantics=("parallel",),
            vmem_limit_bytes=100 * 1024 * 1024,
        ),
    )(y, scale2, shift2)

    return jnp.transpose(out.reshape(N, H2, W2, Cout), (0, 3, 1, 2))


# speedup vs baseline: 1.1043x; 1.1043x over previous
import jax
import jax.numpy as jnp
import numpy as np
from jax.experimental import pallas as pl
from jax.experimental.pallas import tpu as pltpu

BN_EPS = 1e-5


def _conv_stats_kernel(x_ref, w4_ref, sel_ref, y_ref, psum_ref, psq_ref, *,
                       n_pairs, w, c, cout):
    w4 = w4_ref[...]
    sel = sel_ref[...]
    w2h = w // 2
    acc_s = jnp.zeros((w, cout), jnp.float32)
    acc_q = jnp.zeros((w, cout), jnp.float32)
    for u in range(n_pairs // 2):
        rows = [x_ref[0, :, (u * 4 + k) * w:(u * 4 + k + 1) * w]
                for k in range(4)]
        xq4 = jnp.concatenate(rows, axis=0)
        d = jnp.dot(xq4, sel, preferred_element_type=jnp.float32)
        a0, b0 = d[0:c, :w2h], d[0:c, w2h:]
        c0, d0 = d[c:2 * c, :w2h], d[c:2 * c, w2h:]
        a1, b1 = d[2 * c:3 * c, :w2h], d[2 * c:3 * c, w2h:]
        c1, d1 = d[3 * c:, :w2h], d[3 * c:, w2h:]
        cat = jnp.concatenate([
            jnp.concatenate([a0, a1], axis=1),
            jnp.concatenate([b0, b1], axis=1),
            jnp.concatenate([c0, c1], axis=1),
            jnp.concatenate([d0, d1], axis=1),
        ], axis=0)
        y2 = jax.lax.dot_general(
            cat, w4, (((0,), (1,)), ((), ())),
            preferred_element_type=jnp.float32)
        y_ref[0, u * w:(u + 1) * w, :] = y2.astype(y_ref.dtype)
        acc_s = acc_s + y2
        acc_q = acc_q + y2 * y2
    psum_ref[0] = jnp.sum(acc_s, axis=0, keepdims=True)
    psq_ref[0] = jnp.sum(acc_q, axis=0, keepdims=True)


def _bn_relu_kernel(y_ref, scale_ref, shift_ref, o_ref):
    y = y_ref[0].astype(jnp.float32)
    o_ref[0] = jnp.maximum(y * scale_ref[...] + shift_ref[...], 0.0)


def kernel(x, conv_w, conv_b, bn_gamma, bn_beta):
    N, C, H, W = x.shape
    H2, W2 = H // 2, W // 2
    HW = H * W
    P = N * H2 * W2
    Cout = conv_w.shape[0]

    wf = conv_w.reshape(Cout, 4 * C).astype(jnp.float32)
    w0, w1, w2_, w3 = (wf[:, 0:C], wf[:, C:2 * C],
                       wf[:, 2 * C:3 * C], wf[:, 3 * C:])
    wa = 0.5 * (w0 + w1 + w2_ + w3)
    wb = 0.5 * (w0 + w1 - w2_ - w3)
    wc = 0.5 * (w0 - w1 + w2_ - w3)
    wd = 0.5 * (w0 - w1 - w2_ + w3)
    w4 = jnp.concatenate([wa, wb, wc, wd], axis=1)

    W2h = W // 2
    sel_np = np.zeros((W, W), np.float32)
    sel_np[2 * np.arange(W2h), np.arange(W2h)] = 1.0
    sel_np[2 * np.arange(W2h) + 1, W2h + np.arange(W2h)] = 1.0
    sel = jnp.asarray(sel_np)

    npb = H2
    LB = npb * 2 * W
    J = HW // LB

    xr = x.reshape(N, C, HW)

    cparams = pltpu.CompilerParams(
        dimension_semantics=("parallel", "parallel"),
        vmem_limit_bytes=100 * 1024 * 1024,
    )

    y, psum, psq = pl.pallas_call(
        lambda *refs: _conv_stats_kernel(*refs, n_pairs=npb, w=W, c=C,
                                         cout=Cout),
        grid=(N, J),
        in_specs=[
            pl.BlockSpec((1, C, LB), lambda n, j: (n, 0, j)),
            pl.BlockSpec((Cout, 4 * C), lambda n, j: (0, 0)),
            pl.BlockSpec((W, W), lambda n, j: (0, 0)),
        ],
        out_specs=[
            pl.BlockSpec((1, LB // 4, Cout), lambda n, j: (n, j, 0)),
            pl.BlockSpec((1, 1, Cout), lambda n, j: (n * J + j, 0, 0)),
            pl.BlockSpec((1, 1, Cout), lambda n, j: (n * J + j, 0, 0)),
        ],
        out_shape=[
            jax.ShapeDtypeStruct((N, HW // 4, Cout), jnp.bfloat16),
            jax.ShapeDtypeStruct((N * J, 1, Cout), jnp.float32),
            jax.ShapeDtypeStruct((N * J, 1, Cout), jnp.float32),
        ],
        compiler_params=cparams,
    )(xr, w4, sel)

    conv_b = conv_b.astype(jnp.float32)
    sum_y = jnp.sum(psum[:, 0, :], axis=0)
    sum_y2 = jnp.sum(psq[:, 0, :], axis=0)
    mean_nb = sum_y / P
    var = jnp.maximum(sum_y2 / P - mean_nb * mean_nb, 0.0)
    inv_std = jax.lax.rsqrt(var + BN_EPS)
    scale = bn_gamma.astype(jnp.float32) * inv_std
    mean_wb = mean_nb + conv_b
    shift = bn_beta.astype(jnp.float32) + scale * (conv_b - mean_wb)
    scale2 = scale.reshape(1, Cout)
    shift2 = shift.reshape(1, Cout)

    P2 = HW // 4
    out = pl.pallas_call(
        _bn_relu_kernel,
        grid=(N,),
        in_specs=[
            pl.BlockSpec((1, P2, Cout), lambda n: (n, 0, 0)),
            pl.BlockSpec((1, Cout), lambda n: (0, 0)),
            pl.BlockSpec((1, Cout), lambda n: (0, 0)),
        ],
        out_specs=pl.BlockSpec((1, P2, Cout), lambda n: (n, 0, 0)),
        out_shape=jax.ShapeDtypeStruct((N, P2, Cout), jnp.float32),
        compiler_params=pltpu.CompilerParams(
            dimension_semantics=("parallel",),
            vmem_limit_bytes=100 * 1024 * 1024,
        ),
    )(y, scale2, shift2)

    return jnp.transpose(out.reshape(N, H2, W2, Cout), (0, 3, 1, 2))
